# baseline (speedup 1.0000x reference)
import jax
import jax.numpy as jnp
from jax.experimental import pallas as pl

_EMB = 16
_LANES = 128
_VREG = 1024


def _fill_block(pat_ref, out_ref):
    n = out_ref.shape[0]
    pat = pat_ref[0, :]
    block2d = jnp.broadcast_to(pat[None, :], (n // _LANES, _LANES))
    out_ref[:] = block2d.reshape(n)


def _pick_block(total: int, quantum: int, cap: int) -> int:
    best = 0
    b = quantum
    while b <= min(cap, total):
        if total % b == 0:
            best = b
        b += quantum
    return best


def kernel(edge_index, emb_table):
    E = edge_index.shape[1]
    total = E * _EMB
    if total % _VREG == 0:
        pat = jnp.tile(emb_table[0], _LANES // _EMB)[None, :]
        block = _pick_block(total, _VREG, cap=4_194_304)
        out = pl.pallas_call(
            _fill_block,
            grid=(total // block,),
            in_specs=[pl.BlockSpec((1, _LANES), lambda i: (0, 0))],
            out_specs=pl.BlockSpec((block,), lambda i: (i,)),
            out_shape=jax.ShapeDtypeStruct((total,), jnp.float32),
        )(pat)
        return out.reshape(E, _EMB)
    block = _pick_block(E, 8, cap=65_536) or E
    out = pl.pallas_call(
        lambda emb_ref, out_ref: out_ref.__setitem__(
            (slice(None), slice(None)),
            jnp.broadcast_to(emb_ref[0:1, :], out_ref.shape),
        ),
        grid=(E // block,),
        in_specs=[pl.BlockSpec((1, _EMB), lambda i: (0, 0))],
        out_specs=pl.BlockSpec((block, _EMB), lambda i: (i, 0)),
        out_shape=jax.ShapeDtypeStruct((E, _EMB), jnp.float32),
    )(emb_table)
    return out


# speedup vs baseline: 1.1289x; 1.1289x over previous
import jax
import jax.numpy as jnp
from jax.experimental import pallas as pl

_EMB = 16


def _fill_block(emb_ref, out_ref):
    i = pl.program_id(0)

    @pl.when(i < 2)
    def _():
        out_ref[:, :] = jnp.broadcast_to(emb_ref[0:1, :], out_ref.shape)


def _fill_block_always(emb_ref, out_ref):
    out_ref[:, :] = jnp.broadcast_to(emb_ref[0:1, :], out_ref.shape)


def _pick_block(rows: int, quantum: int, cap: int) -> int:
    best = 0
    b = quantum
    while b <= min(cap, rows):
        if rows % b == 0:
            best = b
        b += quantum
    return best


def kernel(edge_index, emb_table):
    E = edge_index.shape[1]
    block = _pick_block(E, 8, cap=40_000)
    if block and E // block >= 4:
        body = _fill_block
    else:
        block = block or E
        body = _fill_block_always
    return pl.pallas_call(
        body,
        grid=(E // block,),
        in_specs=[pl.BlockSpec((1, _EMB), lambda i: (0, 0))],
        out_specs=pl.BlockSpec((block, _EMB), lambda i: (i, 0)),
        out_shape=jax.ShapeDtypeStruct((E, _EMB), jnp.float32),
    )(emb_table)


# speedup vs baseline: 1.1630x; 1.0302x over previous
import functools

import jax
import jax.numpy as jnp
from jax import lax
from jax.experimental import pallas as pl
from jax.experimental.pallas import tpu as pltpu
from jax.experimental.pallas import tpu_sc as plsc

_EMB = 16
_NC, _NS = 2, 16
_NW = _NC * _NS
_CHUNK = 1000
_UNROLL = 8


def _sc_fill(E):
    R = E // _NW
    n_dma = R // _CHUNK
    mesh = plsc.VectorSubcoreMesh(core_axis_name="c", subcore_axis_name="s")

    @functools.partial(
        pl.kernel,
        mesh=mesh,
        out_type=jax.ShapeDtypeStruct((E, _EMB), jnp.float32),
        scratch_types=[
            pltpu.VMEM((1, _EMB), jnp.float32),
            pltpu.VMEM((_CHUNK, _EMB), jnp.float32),
            pltpu.SemaphoreType.DMA,
        ],
    )
    def body(emb_hbm, out_hbm, emb_v, buf, sem):
        wid = lax.axis_index("s") * _NC + lax.axis_index("c")
        base = wid * R
        pltpu.sync_copy(emb_hbm, emb_v)
        row = emb_v[0, :]

        def fill(i, carry):
            for j in range(_UNROLL):
                buf[i * _UNROLL + j, :] = row
            return carry

        lax.fori_loop(0, _CHUNK // _UNROLL, fill, 0)

        copies = [
            pltpu.async_copy(
                buf, out_hbm.at[pl.ds(base + k * _CHUNK, _CHUNK)], sem
            )
            for k in range(n_dma)
        ]
        for c in copies:
            c.wait()

    return body


def kernel(edge_index, emb_table):
    E = edge_index.shape[1]
    if E % (_NW * _CHUNK) == 0:
        return _sc_fill(E)(emb_table)
    block = E
    for b in range(min(65_536, E), 0, -1):
        if E % b == 0:
            block = b
            break
    return pl.pallas_call(
        lambda emb_ref, out_ref: out_ref.__setitem__(
            (slice(None), slice(None)),
            jnp.broadcast_to(emb_ref[0:1, :], out_ref.shape),
        ),
        grid=(E // block,),
        in_specs=[pl.BlockSpec((1, _EMB), lambda i: (0, 0))],
        out_specs=pl.BlockSpec((block, _EMB), lambda i: (i, 0)),
        out_shape=jax.ShapeDtypeStruct((E, _EMB), jnp.float32),
    )(emb_table)
